# baseline (speedup 1.0000x reference)
import functools

import jax
import jax.numpy as jnp
from jax import lax
from jax.experimental import pallas as pl
from jax.experimental.pallas import tpu as pltpu
from jax.experimental.pallas import tpu_sc as plsc

NN = 10000
EE = 160000
IN_D = 256
HID_D = 128
OUT_D = 128
KK = 2

NC, NS, LANES = 2, 16, 16
NW = NC * NS
BB = 128
E_PAD = 163840
EPT = E_PAD // NW
NB = EPT // BB
N_PAD = 10112
RPT = N_PAD // NS

_mesh = plsc.VectorSubcoreMesh(core_axis_name="c", subcore_axis_name="s")


def _zero_rows(buf, nrows, ncols):
    def body(i, _):
        for j in range(ncols // LANES):
            buf[i, pl.ds(j * LANES, LANES)] = jnp.zeros((LANES,), jnp.float32)
        return 0
    lax.fori_loop(0, nrows, body, 0)


def _zero_acc_slice(zbuf, acc, r0):
    off = 0
    while off < RPT:
        ln = min(BB, RPT - off)
        pltpu.sync_copy(zbuf.at[pl.ds(0, ln)], acc.at[pl.ds(r0 + off, ln)])
        off += ln


def _deg_body(cols_hbm, zeros_hbm, out_hbm, acc, ones_v, cols_v):
    c = lax.axis_index("c")
    s = lax.axis_index("s")
    wid = c * NS + s
    pltpu.sync_copy(cols_hbm.at[wid], cols_v)
    pltpu.sync_copy(zeros_hbm, acc.at[pl.ds(s * RPT, RPT)])
    def fill(i, _):
        ones_v[i, pl.ds(0, LANES)] = jnp.ones((LANES,), jnp.float32)
        return 0
    lax.fori_loop(0, BB, fill, 0)
    plsc.subcore_barrier()
    for b in range(NB):
        pltpu.sync_copy(ones_v, acc.at[cols_v.at[b]], add=True)
    plsc.subcore_barrier()
    r0 = s * RPT
    pltpu.sync_copy(acc.at[pl.ds(r0, RPT)], out_hbm.at[c, pl.ds(r0, RPT)])


DEPTH = 2
NQ = NB // DEPTH


def _prop_body(y_hbm, rows_hbm, cols_hbm, zeros_hbm, out_hbm, acc,
               g0, g1, rows_v, cols_v, *sems):
    gb = (g0, g1)
    gsems = sems[:DEPTH]
    ssems = sems[DEPTH:]
    c = lax.axis_index("c")
    s = lax.axis_index("s")
    r0 = s * RPT
    wid = c * NS + s
    pltpu.sync_copy(rows_hbm.at[wid], rows_v)
    pltpu.sync_copy(cols_hbm.at[wid], cols_v)

    for j in range(DEPTH):
        pltpu.async_copy(y_hbm.at[rows_v.at[j]], gb[j], gsems[j])
    pltpu.sync_copy(zeros_hbm, acc.at[pl.ds(r0, RPT)])
    plsc.subcore_barrier()

    def body(q, _):
        for j in range(DEPTH):
            b = q * DEPTH + j
            pltpu.make_async_copy(y_hbm.at[rows_v.at[b]], gb[j],
                                  gsems[j]).wait()
            pltpu.async_copy(gb[j], acc.at[cols_v.at[b]], ssems[j],
                             add=True)
        for j in range(DEPTH):
            b = q * DEPTH + j
            pltpu.make_async_copy(gb[j], acc.at[cols_v.at[b]],
                                  ssems[j]).wait()
            @pl.when(q + 1 < NQ)
            def _():
                pltpu.async_copy(y_hbm.at[rows_v.at[(q + 1) * DEPTH + j]],
                                 gb[j], gsems[j])
        return 0

    lax.fori_loop(0, NQ, body, 0)
    plsc.subcore_barrier()
    pltpu.sync_copy(acc.at[pl.ds(r0, RPT)], out_hbm.at[c, pl.ds(r0, RPT)])


_deg_call = functools.partial(
    pl.kernel,
    _deg_body,
    out_type=jax.ShapeDtypeStruct((NC, N_PAD, HID_D), jnp.float32),
    mesh=_mesh,
    scratch_types=[
        pltpu.VMEM_SHARED((N_PAD, HID_D), jnp.float32),
        pltpu.VMEM((BB, HID_D), jnp.float32),
        pltpu.VMEM((NB, BB), jnp.int32),
    ],
)()

_prop_call = functools.partial(
    pl.kernel,
    _prop_body,
    out_type=jax.ShapeDtypeStruct((NC, N_PAD, HID_D), jnp.float32),
    mesh=_mesh,
    scratch_types=[
        pltpu.VMEM_SHARED((N_PAD, HID_D), jnp.float32),
        pltpu.VMEM((BB, HID_D), jnp.float32),
        pltpu.VMEM((BB, HID_D), jnp.float32),
        pltpu.VMEM((NB, BB), jnp.int32),
        pltpu.VMEM((NB, BB), jnp.int32),
    ] + [pltpu.SemaphoreType.DMA] * (2 * DEPTH),
)()


BR = 1000
GRID = NN // BR


def _dinv_block(dp):
    deg = dp[0, :, 0:1] + dp[1, :, 0:1] + 1.0
    return lax.rsqrt(deg)


def _mlp_kern(x_ref, w1_ref, b1_ref, g_ref, be_ref, w2_ref, b2_ref,
              temp_ref, h_ref, hid_ref):
    h = jnp.dot(x_ref[:], w1_ref[:], preferred_element_type=jnp.float32)
    h = h + b1_ref[:]
    h = 0.5 * h * (1.0 + lax.erf(h * 0.7071067811865476))
    mu = jnp.mean(h, axis=-1, keepdims=True)
    d = h - mu
    var = jnp.mean(d * d, axis=-1, keepdims=True)
    h = d * lax.rsqrt(var + 1e-5) * g_ref[:] + be_ref[:]
    h = jnp.dot(h, w2_ref[:], preferred_element_type=jnp.float32) + b2_ref[:]
    h_ref[:] = h
    hid_ref[:] = temp_ref[0] * h


def _scale_kern(h_ref, dp_ref, y_ref, dinv_ref):
    dinv = _dinv_block(dp_ref[:])
    y_ref[:] = h_ref[:] * dinv
    dinv_ref[:] = jnp.broadcast_to(dinv, (BR, 8))


def _comb_kern(step, pp_ref, y_ref, dinv_in_ref, hid_in_ref, temp_ref,
               *out_refs):
    p = pp_ref[0] + pp_ref[1]
    dinv = dinv_in_ref[:, 0:1]
    hn = dinv * (p + y_ref[:])
    hid = hid_in_ref[:] + temp_ref[step] * hn
    if step < KK:
        out_refs[0][:] = hn * dinv
        out_refs[1][:] = hid
    else:
        out_refs[0][:] = hid


def _row_spec(cols):
    return pl.BlockSpec((BR, cols), lambda i: (i, 0))


_full128 = pl.BlockSpec((HID_D,), lambda i: (0,))
_dp_spec = pl.BlockSpec((NC, BR, HID_D), lambda i: (0, i, 0))
_dinv_spec = pl.BlockSpec((BR, 8), lambda i: (i, 0))
_pp_spec = pl.BlockSpec((NC, BR, HID_D), lambda i: (0, i, 0))
_temp_spec = pl.BlockSpec(memory_space=pltpu.SMEM)
_row128 = _row_spec(HID_D)

_mlp_call = pl.pallas_call(
    _mlp_kern,
    grid=(GRID,),
    in_specs=[
        _row_spec(IN_D),
        pl.BlockSpec((IN_D, HID_D), lambda i: (0, 0)),
        _full128, _full128, _full128,
        pl.BlockSpec((HID_D, OUT_D), lambda i: (0, 0)),
        _full128,
        _temp_spec,
    ],
    out_specs=[_row128, _row128],
    out_shape=[
        jax.ShapeDtypeStruct((NN, HID_D), jnp.float32),
        jax.ShapeDtypeStruct((NN, HID_D), jnp.float32),
    ],
)


_scale_call = pl.pallas_call(
    _scale_kern,
    grid=(GRID,),
    in_specs=[_row128, _dp_spec],
    out_specs=[_row128, _dinv_spec],
    out_shape=[
        jax.ShapeDtypeStruct((NN, HID_D), jnp.float32),
        jax.ShapeDtypeStruct((NN, 8), jnp.float32),
    ],
)


def _make_comb(step):
    last = step == KK
    n_out = 1 if last else 2
    return pl.pallas_call(
        functools.partial(_comb_kern, step),
        grid=(GRID,),
        in_specs=[_pp_spec, _row128, _dinv_spec, _row128, _temp_spec],
        out_specs=[_row128] * n_out,
        out_shape=[jax.ShapeDtypeStruct((NN, HID_D), jnp.float32)] * n_out,
    )


_comb1 = _make_comb(1)
_comb2 = _make_comb(2)


def kernel(x, edge_index, W1, b1, g, beta, W2, b2, temp):
    row = edge_index[0]
    col = edge_index[1]
    pad = E_PAD - EE
    pad_iota = jnp.arange(pad, dtype=jnp.int32)
    rows_p = jnp.concatenate([row, pad_iota % NN])
    cols_p = jnp.concatenate([col, pad_iota % (N_PAD - NN) + NN])
    rows_p = rows_p.reshape(NW, NB, BB)
    cols_p = cols_p.reshape(NW, NB, BB)

    zeros_hbm = jnp.zeros((RPT, HID_D), jnp.float32)

    deg_parts = _deg_call(cols_p, zeros_hbm)
    h, hidden = _mlp_call(x, W1, b1, g, beta, W2, b2, temp)
    y, dinv = _scale_call(h, deg_parts)
    p = _prop_call(y, rows_p, cols_p, zeros_hbm)
    y, hidden = _comb1(p, y, dinv, hidden, temp)
    p = _prop_call(y, rows_p, cols_p, zeros_hbm)
    (hidden,) = _comb2(p, y, dinv, hidden, temp)
    return hidden


# speedup vs baseline: 1.0774x; 1.0774x over previous
import functools

import jax
import jax.numpy as jnp
from jax import lax
from jax.experimental import pallas as pl
from jax.experimental.pallas import tpu as pltpu
from jax.experimental.pallas import tpu_sc as plsc

NN = 10000
EE = 160000
IN_D = 256
HID_D = 128
OUT_D = 128
KK = 2

NC, NS, LANES = 2, 16, 16
NW = NC * NS
BB = 128
E_PAD = 163840
EPT = E_PAD // NW
NB = EPT // BB
N_PAD = 10112
RPT = N_PAD // NS

_mesh = plsc.VectorSubcoreMesh(core_axis_name="c", subcore_axis_name="s")


def _zero_rows(buf, nrows, ncols):
    def body(i, _):
        for j in range(ncols // LANES):
            buf[i, pl.ds(j * LANES, LANES)] = jnp.zeros((LANES,), jnp.float32)
        return 0
    lax.fori_loop(0, nrows, body, 0)


def _zero_acc_slice(zbuf, acc, r0):
    off = 0
    while off < RPT:
        ln = min(BB, RPT - off)
        pltpu.sync_copy(zbuf.at[pl.ds(0, ln)], acc.at[pl.ds(r0 + off, ln)])
        off += ln


def _deg_body(cols_hbm, out_hbm, acc, ones_v, cols_v):
    c = lax.axis_index("c")
    s = lax.axis_index("s")
    wid = c * NS + s
    pltpu.sync_copy(cols_hbm.at[wid], cols_v)
    _zero_rows(ones_v, BB, HID_D)
    _zero_acc_slice(ones_v, acc, s * RPT)
    def fill(i, _):
        ones_v[i, pl.ds(0, LANES)] = jnp.ones((LANES,), jnp.float32)
        return 0
    lax.fori_loop(0, BB, fill, 0)
    plsc.subcore_barrier()
    for b in range(NB):
        pltpu.sync_copy(ones_v, acc.at[cols_v.at[b]], add=True)
    plsc.subcore_barrier()
    r0 = s * RPT
    pltpu.sync_copy(acc.at[pl.ds(r0, RPT)], out_hbm.at[c, pl.ds(r0, RPT)])


DEPTH = 2
NQ = NB // DEPTH


def _prop_body(y_hbm, rows_hbm, cols_hbm, out_hbm, acc,
               g0, g1, rows_v, cols_v, *sems):
    gb = (g0, g1)
    gsems = sems[:DEPTH]
    ssems = sems[DEPTH:]
    c = lax.axis_index("c")
    s = lax.axis_index("s")
    r0 = s * RPT
    wid = c * NS + s
    pltpu.sync_copy(rows_hbm.at[wid], rows_v)
    pltpu.sync_copy(cols_hbm.at[wid], cols_v)
    _zero_rows(g1, BB, HID_D)
    pltpu.async_copy(y_hbm.at[rows_v.at[0]], g0, gsems[0])
    _zero_acc_slice(g1, acc, r0)
    pltpu.async_copy(y_hbm.at[rows_v.at[1]], g1, gsems[1])
    plsc.subcore_barrier()

    def body(q, _):
        for j in range(DEPTH):
            b = q * DEPTH + j
            pltpu.make_async_copy(y_hbm.at[rows_v.at[b]], gb[j],
                                  gsems[j]).wait()
            pltpu.async_copy(gb[j], acc.at[cols_v.at[b]], ssems[j],
                             add=True)
        for j in range(DEPTH):
            b = q * DEPTH + j
            pltpu.make_async_copy(gb[j], acc.at[cols_v.at[b]],
                                  ssems[j]).wait()
            @pl.when(q + 1 < NQ)
            def _():
                pltpu.async_copy(y_hbm.at[rows_v.at[(q + 1) * DEPTH + j]],
                                 gb[j], gsems[j])
        return 0

    lax.fori_loop(0, NQ, body, 0)
    plsc.subcore_barrier()
    pltpu.sync_copy(acc.at[pl.ds(r0, RPT)], out_hbm.at[c, pl.ds(r0, RPT)])


_deg_call = functools.partial(
    pl.kernel,
    _deg_body,
    out_type=jax.ShapeDtypeStruct((NC, N_PAD, HID_D), jnp.float32),
    mesh=_mesh,
    scratch_types=[
        pltpu.VMEM_SHARED((N_PAD, HID_D), jnp.float32),
        pltpu.VMEM((BB, HID_D), jnp.float32),
        pltpu.VMEM((NB, BB), jnp.int32),
    ],
)()

_prop_call = functools.partial(
    pl.kernel,
    _prop_body,
    out_type=jax.ShapeDtypeStruct((NC, N_PAD, HID_D), jnp.float32),
    mesh=_mesh,
    scratch_types=[
        pltpu.VMEM_SHARED((N_PAD, HID_D), jnp.float32),
        pltpu.VMEM((BB, HID_D), jnp.float32),
        pltpu.VMEM((BB, HID_D), jnp.float32),
        pltpu.VMEM((NB, BB), jnp.int32),
        pltpu.VMEM((NB, BB), jnp.int32),
    ] + [pltpu.SemaphoreType.DMA] * (2 * DEPTH),
)()


BR = 1000
GRID = NN // BR


def _dinv_block(dp):
    deg = dp[0, :, 0:1] + dp[1, :, 0:1] + 1.0
    return lax.rsqrt(deg)


def _mlp_kern(x_ref, w1_ref, b1_ref, g_ref, be_ref, w2_ref, b2_ref,
              temp_ref, h_ref, hid_ref):
    h = jnp.dot(x_ref[:], w1_ref[:], preferred_element_type=jnp.float32)
    h = h + b1_ref[:]
    h = 0.5 * h * (1.0 + lax.erf(h * 0.7071067811865476))
    mu = jnp.mean(h, axis=-1, keepdims=True)
    d = h - mu
    var = jnp.mean(d * d, axis=-1, keepdims=True)
    h = d * lax.rsqrt(var + 1e-5) * g_ref[:] + be_ref[:]
    h = jnp.dot(h, w2_ref[:], preferred_element_type=jnp.float32) + b2_ref[:]
    h_ref[:] = h
    hid_ref[:] = temp_ref[0] * h


def _scale_kern(h_ref, dp_ref, y_ref, dinv_ref):
    dinv = _dinv_block(dp_ref[:])
    y_ref[:] = h_ref[:] * dinv
    dinv_ref[:] = jnp.broadcast_to(dinv, (BR, 8))


def _comb_kern(step, pp_ref, y_ref, dinv_in_ref, hid_in_ref, temp_ref,
               *out_refs):
    p = pp_ref[0] + pp_ref[1]
    dinv = dinv_in_ref[:, 0:1]
    hn = dinv * (p + y_ref[:])
    hid = hid_in_ref[:] + temp_ref[step] * hn
    if step < KK:
        out_refs[0][:] = hn * dinv
        out_refs[1][:] = hid
    else:
        out_refs[0][:] = hid


def _row_spec(cols):
    return pl.BlockSpec((BR, cols), lambda i: (i, 0))


_full128 = pl.BlockSpec((HID_D,), lambda i: (0,))
_dp_spec = pl.BlockSpec((NC, BR, HID_D), lambda i: (0, i, 0))
_dinv_spec = pl.BlockSpec((BR, 8), lambda i: (i, 0))
_pp_spec = pl.BlockSpec((NC, BR, HID_D), lambda i: (0, i, 0))
_temp_spec = pl.BlockSpec(memory_space=pltpu.SMEM)
_row128 = _row_spec(HID_D)

_mlp_call = pl.pallas_call(
    _mlp_kern,
    grid=(GRID,),
    in_specs=[
        _row_spec(IN_D),
        pl.BlockSpec((IN_D, HID_D), lambda i: (0, 0)),
        _full128, _full128, _full128,
        pl.BlockSpec((HID_D, OUT_D), lambda i: (0, 0)),
        _full128,
        _temp_spec,
    ],
    out_specs=[_row128, _row128],
    out_shape=[
        jax.ShapeDtypeStruct((NN, HID_D), jnp.float32),
        jax.ShapeDtypeStruct((NN, HID_D), jnp.float32),
    ],
)


_scale_call = pl.pallas_call(
    _scale_kern,
    grid=(GRID,),
    in_specs=[_row128, _dp_spec],
    out_specs=[_row128, _dinv_spec],
    out_shape=[
        jax.ShapeDtypeStruct((NN, HID_D), jnp.float32),
        jax.ShapeDtypeStruct((NN, 8), jnp.float32),
    ],
)


def _make_comb(step):
    last = step == KK
    n_out = 1 if last else 2
    return pl.pallas_call(
        functools.partial(_comb_kern, step),
        grid=(GRID,),
        in_specs=[_pp_spec, _row128, _dinv_spec, _row128, _temp_spec],
        out_specs=[_row128] * n_out,
        out_shape=[jax.ShapeDtypeStruct((NN, HID_D), jnp.float32)] * n_out,
    )


_comb1 = _make_comb(1)
_comb2 = _make_comb(2)


def kernel(x, edge_index, W1, b1, g, beta, W2, b2, temp):
    row = edge_index[0]
    col = edge_index[1]
    pad = E_PAD - EE
    pad_iota = jnp.arange(pad, dtype=jnp.int32)
    rows_p = jnp.concatenate([row, pad_iota % NN])
    cols_p = jnp.concatenate([col, pad_iota % (N_PAD - NN) + NN])
    rows_p = rows_p.reshape(NW, NB, BB)
    cols_p = cols_p.reshape(NW, NB, BB)

    deg_parts = _deg_call(cols_p)
    h, hidden = _mlp_call(x, W1, b1, g, beta, W2, b2, temp)
    y, dinv = _scale_call(h, deg_parts)
    p = _prop_call(y, rows_p, cols_p)
    y, hidden = _comb1(p, y, dinv, hidden, temp)
    p = _prop_call(y, rows_p, cols_p)
    (hidden,) = _comb2(p, y, dinv, hidden, temp)
    return hidden
